# baseline (speedup 1.0000x reference)
import jax
import jax.numpy as jnp
from jax.experimental import pallas as pl

N = 256
OUT_W = N * (N + 1) // 2
BATCH_BLK = 64


def _seg_off(i):
    return i * N - i * (i - 1) // 2


def _body(x_ref, o_ref):
    for tr in range(N // 8):
        blk = jnp.swapaxes(x_ref[:, 8 * tr : 8 * tr + 8, :], 0, 1)
        for s in range(8):
            i = 8 * tr + s
            m = N - i
            o_ref[:, pl.ds(_seg_off(i), m)] = blk[s, :, i:]


def kernel(input):
    B = input.shape[0]
    out = pl.pallas_call(
        _body,
        grid=(B // BATCH_BLK,),
        in_specs=[pl.BlockSpec((BATCH_BLK, N, N), lambda b: (b, 0, 0))],
        out_specs=pl.BlockSpec((BATCH_BLK, OUT_W), lambda b: (b, 0)),
        out_shape=jax.ShapeDtypeStruct((B, OUT_W), input.dtype),
    )(input)
    return out


# speedup vs baseline: 1.1422x; 1.1422x over previous
import jax
import jax.numpy as jnp
from jax.experimental import pallas as pl

N = 256
OUT_W = N * (N + 1) // 2
BATCH_BLK = 64
H = N // 2


def _seg_off(i):
    return i * N - i * (i - 1) // 2


def _body(xa_ref, xc_ref, xd_ref, o_ref):
    for tr in range(N // 8):
        if tr < 16:
            blk = jnp.swapaxes(xa_ref[:, 8 * tr : 8 * tr + 8, :], 0, 1)
            col_base = 0
        elif tr < 24:
            blk = jnp.swapaxes(xc_ref[:, 8 * tr - 128 : 8 * tr - 120, :], 0, 1)
            col_base = H
        else:
            blk = jnp.swapaxes(xd_ref[:, 8 * tr - 192 : 8 * tr - 184, :], 0, 1)
            col_base = H
        for s in range(8):
            i = 8 * tr + s
            m = N - i
            o_ref[:, pl.ds(_seg_off(i), m)] = blk[s, :, i - col_base :]


def kernel(input):
    B = input.shape[0]
    BB = BATCH_BLK
    out = pl.pallas_call(
        _body,
        grid=(B // BB,),
        in_specs=[
            pl.BlockSpec((BB, 128, 256), lambda b: (b, 0, 0)),
            pl.BlockSpec((BB, 64, 128), lambda b: (b, 2, 1)),
            pl.BlockSpec((BB, 64, 128), lambda b: (b, 3, 1)),
        ],
        out_specs=pl.BlockSpec((BB, OUT_W), lambda b: (b, 0)),
        out_shape=jax.ShapeDtypeStruct((B, OUT_W), input.dtype),
    )(input, input, input)
    return out
